# baseline (speedup 1.0000x reference)
import math

import jax
import jax.numpy as jnp
from jax.experimental import pallas as pl
from jax.experimental.pallas import tpu as pltpu

MAX_L = 8
K = (MAX_L + 1) ** 2
LANES = 128
CHUNK = 64
SUBS = 8
NBUF = 2
SQRT2 = math.sqrt(2.0)
Y00 = math.sqrt(1.0 / (4.0 * math.pi))


def _chain(x, y, z, emit):
    rho2 = x * x + y * y
    r2 = rho2 + z * z
    ct = jnp.clip(z * jax.lax.rsqrt(r2), -1.0, 1.0)
    st = jnp.sqrt(jnp.maximum(1.0 - ct * ct, 0.0))
    safe = rho2 > 0.0
    inv_rho = jax.lax.rsqrt(jnp.where(safe, rho2, 1.0))
    ca = jnp.where(safe, x * inv_rho, 1.0)
    sa = jnp.where(safe, y * inv_rho, 0.0)

    pmm = jnp.full_like(x, Y00)
    cmv = smv = None
    c2m = s2m = None
    for m in range(MAX_L + 1):
        if m > 0:
            pmm = (-math.sqrt((2 * m + 1) / (2.0 * m)) * st) * pmm
            if m == 1:
                cmv, smv = ca, sa
            else:
                cmv, smv = cmv * ca - smv * sa, smv * ca + cmv * sa
            c2m = SQRT2 * cmv
            s2m = SQRT2 * smv

        def em(l, p):
            if m == 0:
                emit(l * (l + 1), p)
            else:
                emit(l * (l + 1) + m, p * c2m)
                emit(l * (l + 1) - m, p * s2m)

        em(m, pmm)
        if m < MAX_L:
            p_prev2 = pmm
            p_prev = (math.sqrt(2 * m + 3) * ct) * pmm
            em(m + 1, p_prev)
            for l in range(m + 2, MAX_L + 1):
                a = math.sqrt((4.0 * l * l - 1.0) / (l * l - m * m))
                b = -math.sqrt(((2 * l + 1.0) * ((l - 1) ** 2 - m * m))
                               / ((2 * l - 3.0) * (l * l - m * m)))
                p = a * (ct * p_prev) + b * p_prev2
                em(l, p)
                p_prev2, p_prev = p_prev, p


def _compute_chunk(c_rows, out_view):
    lane = jax.lax.broadcasted_iota(jnp.int32, (8, LANES), 1)
    for g in range(CHUNK // 8):
        c8 = c_rows[g * 8:(g + 1) * 8, :]
        xs = [jnp.broadcast_to(c8[:, k:k + 1], (8, LANES)) for k in range(6)]

        box = {}

        def emit2(idx, col):
            if not box:
                box['psi2'] = col
            else:
                box['psi2'] = jnp.where(lane == idx, col, box['psi2'])

        _chain(xs[3], xs[4], xs[5], emit2)
        psi2 = box['psi2'][:, :K]

        def emit1(idx, col):
            out_view[g * 8:(g + 1) * 8, idx * K:(idx + 1) * K] = \
                col[:, :K] * psi2

        _chain(xs[0], xs[1], xs[2], emit1)


def _tph_kernel(c_ref, o_ref, scr, sem):
    pid = pl.program_id(0)
    rows_per_core = c_ref.shape[0]
    n_sub = rows_per_core // CHUNK
    dma_rows = SUBS * CHUNK
    core_base = pid * rows_per_core

    def body(k, carry):
        sub = jax.lax.rem(k, SUBS)
        sup = jax.lax.div(k, SUBS)
        slot = jax.lax.rem(sup, NBUF)

        @pl.when((sub == 0) & (sup >= NBUF))
        def _():
            pltpu.make_async_copy(scr.at[slot], scr.at[slot],
                                  sem.at[slot]).wait()

        row0 = pl.multiple_of(sub * CHUNK, CHUNK)
        _compute_chunk(c_ref[pl.ds(k * CHUNK, CHUNK), :],
                       scr.at[slot].at[pl.ds(row0, CHUNK), :])

        @pl.when(sub == SUBS - 1)
        def _():
            dst = pl.ds(pl.multiple_of(core_base + sup * dma_rows, dma_rows),
                        dma_rows)
            pltpu.make_async_copy(scr.at[slot], o_ref.at[dst, :],
                                  sem.at[slot]).start()
        return carry

    jax.lax.fori_loop(0, n_sub, body, 0)
    for s in range(NBUF):
        pltpu.make_async_copy(scr.at[s], scr.at[s], sem.at[s]).wait()


def _tph_call(coordinates, interpret=False):
    n = coordinates.shape[0]
    return pl.pallas_call(
        _tph_kernel,
        grid=(2,),
        in_specs=[pl.BlockSpec((n // 2, 6), lambda c: (c, 0))],
        out_specs=pl.BlockSpec(memory_space=pl.ANY),
        out_shape=jax.ShapeDtypeStruct((n, K * K), jnp.float32),
        scratch_shapes=[
            pltpu.VMEM((NBUF, SUBS * CHUNK, K * K), jnp.float32),
            pltpu.SemaphoreType.DMA((NBUF,)),
        ],
        compiler_params=pltpu.CompilerParams(
            dimension_semantics=("parallel",),
            vmem_limit_bytes=56 * 1024 * 1024,
        ),
        interpret=interpret,
    )(coordinates)


@jax.jit
def kernel(coordinates):
    return _tph_call(coordinates)


# speedup vs baseline: 1.1270x; 1.1270x over previous
import math

import jax
import jax.numpy as jnp
from jax.experimental import pallas as pl
from jax.experimental.pallas import tpu as pltpu

MAX_L = 8
K = (MAX_L + 1) ** 2
LANES = 128
CHUNK = 256
NBUF = 2
SQRT2 = math.sqrt(2.0)
Y00 = math.sqrt(1.0 / (4.0 * math.pi))


def _sph_cols(x, y, z):
    rho2 = x * x + y * y
    r2 = rho2 + z * z
    ct = jnp.clip(z * jax.lax.rsqrt(r2), -1.0, 1.0)
    st = jnp.sqrt(jnp.maximum(1.0 - ct * ct, 0.0))
    safe = rho2 > 0.0
    inv_rho = jax.lax.rsqrt(jnp.where(safe, rho2, 1.0))
    ca = jnp.where(safe, x * inv_rho, 1.0)
    sa = jnp.where(safe, y * inv_rho, 0.0)

    cm = {1: ca}
    sm = {1: sa}
    for m in range(1, MAX_L):
        cm[m + 1] = cm[m] * ca - sm[m] * sa
        sm[m + 1] = sm[m] * ca + cm[m] * sa
    c2 = {m: SQRT2 * cm[m] for m in cm}
    s2 = {m: SQRT2 * sm[m] for m in sm}

    P = {(0, 0): jnp.full_like(ct, Y00)}
    for m in range(1, MAX_L + 1):
        c = -math.sqrt((2 * m + 1) / (2.0 * m))
        P[(m, m)] = (c * st) * P[(m - 1, m - 1)]
    for m in range(0, MAX_L):
        c = math.sqrt(2 * m + 3)
        P[(m + 1, m)] = (c * ct) * P[(m, m)]
    for m in range(0, MAX_L + 1):
        for l in range(m + 2, MAX_L + 1):
            a = math.sqrt((4.0 * l * l - 1.0) / (l * l - m * m))
            b = -math.sqrt(((2 * l + 1.0) * ((l - 1) ** 2 - m * m))
                           / ((2 * l - 3.0) * (l * l - m * m)))
            P[(l, m)] = a * ct * P[(l - 1, m)] + b * P[(l - 2, m)]

    cols = [None] * K
    for l in range(MAX_L + 1):
        for m in range(-l, l + 1):
            am = abs(m)
            if m > 0:
                y_lm = P[(l, am)] * c2[m]
            elif m == 0:
                y_lm = P[(l, 0)]
            else:
                y_lm = P[(l, am)] * s2[am]
            cols[l * (l + 1) + m] = y_lm
    return cols


def _compute_chunk(c_rows, out_view):
    rep = [jnp.broadcast_to(c_rows[:, k:k + 1], (CHUNK, LANES))
           for k in range(6)]
    cols1 = _sph_cols(rep[0], rep[1], rep[2])
    cols2 = _sph_cols(rep[3], rep[4], rep[5])

    lane = jax.lax.broadcasted_iota(jnp.int32, (CHUNK, K), 1)
    psi2 = cols2[K - 1][:, :K]
    for j in range(K - 2, -1, -1):
        psi2 = jnp.where(lane == j, cols2[j][:, :K], psi2)

    for i in range(K):
        out_view[:, i * K:(i + 1) * K] = cols1[i][:, :K] * psi2


def _tph_kernel(c_ref, o_ref, scr, sem):
    pid = pl.program_id(0)
    rows_per_core = c_ref.shape[0]
    n_chunks = rows_per_core // CHUNK
    core_base = pid * rows_per_core

    def body(k, carry):
        slot = jax.lax.rem(k, NBUF)

        @pl.when(k >= NBUF)
        def _():
            pltpu.make_async_copy(scr.at[slot], scr.at[slot],
                                  sem.at[slot]).wait()

        _compute_chunk(c_ref[pl.ds(k * CHUNK, CHUNK), :], scr.at[slot])
        dst = pl.ds(pl.multiple_of(core_base + k * CHUNK, CHUNK), CHUNK)
        pltpu.make_async_copy(scr.at[slot], o_ref.at[dst, :],
                              sem.at[slot]).start()
        return carry

    jax.lax.fori_loop(0, n_chunks, body, 0)
    for s in range(NBUF):
        pltpu.make_async_copy(scr.at[s], scr.at[s], sem.at[s]).wait()


def _tph_call(coordinates, interpret=False):
    n = coordinates.shape[0]
    return pl.pallas_call(
        _tph_kernel,
        grid=(2,),
        in_specs=[pl.BlockSpec((n // 2, 6), lambda c: (c, 0))],
        out_specs=pl.BlockSpec(memory_space=pl.ANY),
        out_shape=jax.ShapeDtypeStruct((n, K * K), jnp.float32),
        scratch_shapes=[
            pltpu.VMEM((NBUF, CHUNK, K * K), jnp.float32),
            pltpu.SemaphoreType.DMA((NBUF,)),
        ],
        compiler_params=pltpu.CompilerParams(
            dimension_semantics=("parallel",),
            vmem_limit_bytes=56 * 1024 * 1024,
        ),
        interpret=interpret,
    )(coordinates)


@jax.jit
def kernel(coordinates):
    return _tph_call(coordinates)
